# baseline (speedup 1.0000x reference)
import functools

import jax
import jax.numpy as jnp
from jax import lax
from jax.experimental import pallas as pl
from jax.experimental.pallas import tpu as pltpu
from jax.experimental.pallas import tpu_sc as plsc

B, L, A, K = 4, 1024, 4, 30
BL = B * L
DIM_NODES, DIM_EDGES = 256, 128
RB = 256
RB2 = 64
NW = 32
RPT = BL // NW
BR = 8
NBATCH = RPT // BR
F32 = jnp.float32


def _enc_body(xt_ref, xr_ref, c_ref, wn_ref, bn_ref,
              d_ref, nodeh_ref, maski_ref):
    i = pl.program_id(1)
    xt = xt_ref[0]
    xr = xr_ref[0]

    cax = (((xt[0:1] + xt[3:4]) + xt[6:7]) + xt[9:10]) * 0.25
    cay = (((xt[1:2] + xt[4:5]) + xt[7:8]) + xt[10:11]) * 0.25
    caz = (((xt[2:3] + xt[5:6]) + xt[8:9]) + xt[11:12]) * 0.25
    crx = (((xr[:, 0:1] + xr[:, 3:4]) + xr[:, 6:7]) + xr[:, 9:10]) * 0.25
    cry = (((xr[:, 1:2] + xr[:, 4:5]) + xr[:, 7:8]) + xr[:, 10:11]) * 0.25
    crz = (((xr[:, 2:3] + xr[:, 5:6]) + xr[:, 8:9]) + xr[:, 11:12]) * 0.25

    dx = crx - cax
    dy = cry - cay
    dz = crz - caz
    dist = jnp.sqrt((dx * dx + dy * dy) + dz * dz + 1e-8)
    colj = lax.broadcasted_iota(jnp.int32, (RB, L), 1)
    rowg = i * RB + lax.broadcasted_iota(jnp.int32, (RB, L), 0)
    d_ref[...] = dist + jnp.where(colj == rowg, F32(1e6), F32(0.0))

    cols = []
    for (a, b) in ((0, 1), (0, 2), (0, 3), (1, 2), (1, 3), (2, 3)):
        pdx = xr[:, 3 * a:3 * a + 1] - xr[:, 3 * b:3 * b + 1]
        pdy = xr[:, 3 * a + 1:3 * a + 2] - xr[:, 3 * b + 1:3 * b + 2]
        pdz = xr[:, 3 * a + 2:3 * a + 3] - xr[:, 3 * b + 2:3 * b + 3]
        pd2 = (pdx * pdx + pdy * pdy) + pdz * pdz
        cols.append(jnp.log1p(jnp.sqrt(pd2 + 1e-8)))
    nf = jnp.concatenate(cols + [jnp.zeros((RB, 2), F32)], axis=1)

    mask = (c_ref[0] > 0).astype(F32)
    nh = jnp.dot(nf, wn_ref[...], preferred_element_type=F32) + bn_ref[...]
    nodeh_ref[0] = nh * mask
    maski_ref[0] = mask


def _encode_nodes(Xf, Xt, C3, Wn_pad, bn2):
    grid = (B, L // RB)
    nblk = L // RB
    return pl.pallas_call(
        _enc_body,
        grid=grid,
        in_specs=[
            pl.BlockSpec((1, 12, L), lambda b, i: (b, 0, 0)),
            pl.BlockSpec((1, RB, 12), lambda b, i: (b, i, 0)),
            pl.BlockSpec((1, RB, 1), lambda b, i: (b, i, 0)),
            pl.BlockSpec((8, DIM_NODES), lambda b, i: (0, 0)),
            pl.BlockSpec((1, DIM_NODES), lambda b, i: (0, 0)),
        ],
        out_specs=[
            pl.BlockSpec((RB, L), lambda b, i: (b * nblk + i, 0)),
            pl.BlockSpec((1, RB, DIM_NODES), lambda b, i: (b, i, 0)),
            pl.BlockSpec((1, RB, 1), lambda b, i: (b, i, 0)),
        ],
        out_shape=[
            jax.ShapeDtypeStruct((BL, L), F32),
            jax.ShapeDtypeStruct((B, L, DIM_NODES), F32),
            jax.ShapeDtypeStruct((B, L, 1), F32),
        ],
    )(Xt, Xf, C3, Wn_pad, bn2)


def _sc_topk_gather(D, table):
    mesh = plsc.VectorSubcoreMesh(core_axis_name="c", subcore_axis_name="s")

    @functools.partial(
        pl.kernel, mesh=mesh,
        compiler_params=pltpu.CompilerParams(
            use_tc_tiling_on_sc=True, needs_layout_passes=False),
        out_type=[
            jax.ShapeDtypeStruct((BL, 32), jnp.int32),
            jax.ShapeDtypeStruct((BL * 32, 128), F32),
        ],
        scratch_types=[
            pltpu.VMEM((BR, L), F32),
            pltpu.VMEM((BR, L), F32),
            pltpu.VMEM((1056,), F32),
            pltpu.VMEM((1056,), jnp.int32),
            pltpu.VMEM((RPT, 32), jnp.int32),
            pltpu.VMEM((BR * 32,), jnp.int32),
            pltpu.VMEM((BR * 32,), jnp.int32),
            pltpu.VMEM((BR * 32, 128), F32),
            pltpu.VMEM((BR * 32, 128), F32),
            pltpu.SemaphoreType.DMA,
            pltpu.SemaphoreType.DMA,
            pltpu.SemaphoreType.DMA,
            pltpu.SemaphoreType.DMA,
            pltpu.SemaphoreType.DMA,
            pltpu.SemaphoreType.DMA,
        ],
    )
    def k(d_hbm, t_hbm, eidx_hbm, xj_hbm, db0, db1, cval, cidx, ebuf,
          gb0, gb1, xb0, xb1, sd0, sd1, sg0, sg1, so0, so1):
        wid = lax.axis_index("s") * 2 + lax.axis_index("c")
        r0 = wid * RPT
        base = (r0 // L) * L
        lanes = lax.broadcasted_iota(jnp.int32, (16,), 0)
        infv = jnp.full((16,), jnp.inf, F32)
        bigi = jnp.full((16,), jnp.int32(2 ** 30), jnp.int32)
        dbufs = (db0, db1)
        dsems = (sd0, sd1)
        gbufs = (gb0, gb1)
        xbufs = (xb0, xb1)
        gsems = (sg0, sg1)
        osems = (so0, so1)

        def topk_row(r, dbuf):
            def p1(i, carry):
                m1, m2 = carry
                v = dbuf[r, pl.ds(i * 16, 16)]
                m2n = jnp.minimum(m2, jnp.maximum(m1, v))
                m1n = jnp.minimum(m1, v)
                return m1n, m2n
            m1, m2 = lax.fori_loop(0, L // 16, p1, (infv, infv))
            t = jnp.maximum(jnp.max(m1), jnp.max(m2))

            def p2(i, cnt):
                v = dbuf[r, pl.ds(i * 16, 16)]
                msk = v <= t
                idxv = i * 16 + lanes
                plsc.store_compressed(cval.at[pl.ds(cnt, 16)], v, mask=msk)
                plsc.store_compressed(cidx.at[pl.ds(cnt, 16)], idxv, mask=msk)
                return cnt + jnp.max(plsc.all_reduce_population_count(msk))
            cnt = lax.fori_loop(0, L // 16, p2, jnp.int32(0))
            cval[pl.ds(cnt, 16)] = infv
            nv = (cnt + 15) // 16

            own = r0 + r - base

            def ext(kk, carry):
                e0, e1 = carry

                def scan(c, sc):
                    bv, bi = sc
                    v = cval[pl.ds(c * 16, 16)]
                    ix = cidx[pl.ds(c * 16, 16)]
                    take = (v < bv) | ((v == bv) & (ix < bi))
                    return (jnp.where(take, v, bv), jnp.where(take, ix, bi))
                bv, bi = lax.fori_loop(0, nv, scan, (infv, bigi))
                m = jnp.min(bv)
                mi = jnp.min(jnp.where(bv == m, bi, jnp.int32(2 ** 30)))

                def inval(c, _):
                    v = cval[pl.ds(c * 16, 16)]
                    ix = cidx[pl.ds(c * 16, 16)]
                    cval[pl.ds(c * 16, 16)] = jnp.where(ix == mi, jnp.inf, v)
                    return 0
                lax.fori_loop(0, nv, inval, 0)
                e0 = jnp.where(lanes == kk, mi, e0)
                e1 = jnp.where(lanes == kk - 16, mi, e1)
                return e0, e1

            owns = jnp.full((16,), own, jnp.int32)
            e0, e1 = lax.fori_loop(0, K, ext, (owns, owns))
            return e0, e1

        dcp = pltpu.async_copy(d_hbm.at[pl.ds(r0, BR)], db0, sd0)
        outcp = [None, None]
        for bt in range(NBATCH):
            par = bt % 2
            if bt + 1 < NBATCH:
                ncp = pltpu.async_copy(
                    d_hbm.at[pl.ds(r0 + (bt + 1) * BR, BR)],
                    dbufs[(bt + 1) % 2], dsems[(bt + 1) % 2])
            dcp.wait()
            if outcp[par] is not None:
                outcp[par].wait()

            dbuf = dbufs[par]
            gbuf = gbufs[par]

            def row_fn(r, _):
                e0, e1 = topk_row(r, dbuf)
                row = bt * BR + r
                ebuf[row, pl.ds(0, 16)] = e0
                ebuf[row, pl.ds(16, 16)] = e1
                gbuf[pl.ds(r * 32, 16)] = e0 + base
                gbuf[pl.ds(r * 32 + 16, 16)] = e1 + base
                return 0
            lax.fori_loop(0, BR, row_fn, 0)

            g0 = pltpu.async_copy(t_hbm.at[gbuf.at[pl.ds(0, 128)]],
                                  xbufs[par].at[pl.ds(0, 128)], gsems[par])
            g1 = pltpu.async_copy(t_hbm.at[gbuf.at[pl.ds(128, 128)]],
                                  xbufs[par].at[pl.ds(128, 128)], gsems[par])
            g0.wait()
            g1.wait()
            outcp[par] = pltpu.async_copy(
                xbufs[par],
                xj_hbm.at[pl.ds((r0 + bt * BR) * 32, BR * 32)], osems[par])
            if bt + 1 < NBATCH:
                dcp = ncp
        for oc in outcp:
            if oc is not None:
                oc.wait()
        pltpu.sync_copy(ebuf, eidx_hbm.at[pl.ds(r0, RPT)])

    return k(D, table)


def _edge_body(ti_ref, xj_ref, w16_ref, w17_ref, be_ref, eh_ref, mij_ref):
    xi = ti_ref[...]
    xj = xj_ref[...]
    xib = jnp.broadcast_to(xi[:, None, :], (RB2, 32, 128))

    diff = xib[:, :, 0:48] - xj[:, :, 48:96]
    dx = diff[:, :, 0:16]
    dy = diff[:, :, 16:32]
    dz = diff[:, :, 32:48]
    d2 = (dx * dx + dy * dy) + dz * dz
    dcr = jnp.log1p(jnp.sqrt(d2 + 1e-8))
    offv = jnp.tanh((xj[:, :, 97:98] - xib[:, :, 97:98]) * F32(1.0 / 32.0))

    eh = jnp.dot(dcr.reshape(RB2 * 32, 16), w16_ref[...],
                 preferred_element_type=F32).reshape(RB2, 32, DIM_EDGES)
    eh = eh + offv * w17_ref[...].reshape(1, 1, DIM_EDGES)
    eh = eh + be_ref[...].reshape(1, 1, DIM_EDGES)
    mij = xib[:, :, 96:97] * xj[:, :, 96:97]
    eh_ref[...] = (eh * mij)[:, :K, :]
    mij_ref[...] = mij[:, :K, :]


def _encode_edges(table, Xj, W16, W17, be2):
    grid = (BL // RB2,)
    return pl.pallas_call(
        _edge_body,
        grid=grid,
        in_specs=[
            pl.BlockSpec((RB2, 128), lambda i: (i, 0)),
            pl.BlockSpec((RB2, 32, 128), lambda i: (i, 0, 0)),
            pl.BlockSpec((16, DIM_EDGES), lambda i: (0, 0)),
            pl.BlockSpec((1, DIM_EDGES), lambda i: (0, 0)),
            pl.BlockSpec((1, DIM_EDGES), lambda i: (0, 0)),
        ],
        out_specs=[
            pl.BlockSpec((RB2, K, DIM_EDGES), lambda i: (i, 0, 0)),
            pl.BlockSpec((RB2, K, 1), lambda i: (i, 0, 0)),
        ],
        out_shape=[
            jax.ShapeDtypeStruct((BL, K, DIM_EDGES), F32),
            jax.ShapeDtypeStruct((BL, K, 1), F32),
        ],
    )(table, Xj, W16, W17, be2)


def kernel(X, C, W_node, b_node, W_edge, b_edge):
    Xf = X.reshape(B, L, A * 3)
    Xt = jnp.transpose(Xf, (0, 2, 1))
    C3 = C.reshape(B, L, 1)
    Wn_pad = jnp.concatenate([W_node, jnp.zeros((2, DIM_NODES), F32)], axis=0)
    bn2 = b_node.reshape(1, DIM_NODES)

    D, node_h, mask_i3 = _encode_nodes(Xf, Xt, C3, Wn_pad, bn2)
    mask_i = mask_i3.reshape(B, L)

    Xc = jnp.transpose(X, (0, 1, 3, 2))
    ti48 = jnp.broadcast_to(Xc[:, :, :, :, None],
                            (B, L, 3, 4, 4)).reshape(B, L, 48)
    tj48 = jnp.broadcast_to(Xc[:, :, :, None, :],
                            (B, L, 3, 4, 4)).reshape(B, L, 48)
    lvals = jnp.broadcast_to(
        jnp.arange(L, dtype=F32)[None, :, None], (B, L, 1))
    pad = jnp.zeros((B, L, 30), F32)
    table = jnp.concatenate([ti48, tj48, mask_i3, lvals, pad],
                            axis=2).reshape(BL, 128)

    eidx32, xj = _sc_topk_gather(D, table)
    edge_idx = eidx32[:, :K].reshape(B, L, K)
    Xj = xj.reshape(BL, 32, 128)

    W16 = W_edge[0:16]
    W17 = W_edge[16:17]
    be2 = b_edge.reshape(1, DIM_EDGES)
    eh_flat, mij_flat = _encode_edges(table, Xj, W16, W17, be2)

    edge_h = eh_flat.reshape(B, L, K, DIM_EDGES)
    mask_ij = mij_flat.reshape(B, L, K)
    return node_h, edge_h, edge_idx, mask_i, mask_ij


# speedup vs baseline: 1.1738x; 1.1738x over previous
import functools

import jax
import jax.numpy as jnp
from jax import lax
from jax.experimental import pallas as pl
from jax.experimental.pallas import tpu as pltpu
from jax.experimental.pallas import tpu_sc as plsc

B, L, A, K = 4, 1024, 4, 30
BL = B * L
DIM_NODES, DIM_EDGES = 256, 128
RB = 256
RB2 = 64
NW = 32
RPT = BL // NW
BR = 8
NBATCH = RPT // BR
F32 = jnp.float32


def _enc_body(xt_ref, xr_ref, c_ref, wn_ref, bn_ref,
              d_ref, nodeh_ref, maski_ref):
    i = pl.program_id(1)
    xt = xt_ref[0]
    xr = xr_ref[0]

    cax = (((xt[0:1] + xt[3:4]) + xt[6:7]) + xt[9:10]) * 0.25
    cay = (((xt[1:2] + xt[4:5]) + xt[7:8]) + xt[10:11]) * 0.25
    caz = (((xt[2:3] + xt[5:6]) + xt[8:9]) + xt[11:12]) * 0.25
    crx = (((xr[:, 0:1] + xr[:, 3:4]) + xr[:, 6:7]) + xr[:, 9:10]) * 0.25
    cry = (((xr[:, 1:2] + xr[:, 4:5]) + xr[:, 7:8]) + xr[:, 10:11]) * 0.25
    crz = (((xr[:, 2:3] + xr[:, 5:6]) + xr[:, 8:9]) + xr[:, 11:12]) * 0.25

    dx = crx - cax
    dy = cry - cay
    dz = crz - caz
    dist = jnp.sqrt((dx * dx + dy * dy) + dz * dz + 1e-8)
    colj = lax.broadcasted_iota(jnp.int32, (RB, L), 1)
    rowg = i * RB + lax.broadcasted_iota(jnp.int32, (RB, L), 0)
    d_ref[...] = dist + jnp.where(colj == rowg, F32(1e6), F32(0.0))

    cols = []
    for (a, b) in ((0, 1), (0, 2), (0, 3), (1, 2), (1, 3), (2, 3)):
        pdx = xr[:, 3 * a:3 * a + 1] - xr[:, 3 * b:3 * b + 1]
        pdy = xr[:, 3 * a + 1:3 * a + 2] - xr[:, 3 * b + 1:3 * b + 2]
        pdz = xr[:, 3 * a + 2:3 * a + 3] - xr[:, 3 * b + 2:3 * b + 3]
        pd2 = (pdx * pdx + pdy * pdy) + pdz * pdz
        cols.append(jnp.log1p(jnp.sqrt(pd2 + 1e-8)))
    nf = jnp.concatenate(cols + [jnp.zeros((RB, 2), F32)], axis=1)

    mask = (c_ref[0] > 0).astype(F32)
    nh = jnp.dot(nf, wn_ref[...], preferred_element_type=F32) + bn_ref[...]
    nodeh_ref[0] = nh * mask
    maski_ref[0] = mask


def _encode_nodes(Xf, Xt, C3, Wn_pad, bn2):
    grid = (B, L // RB)
    nblk = L // RB
    return pl.pallas_call(
        _enc_body,
        grid=grid,
        in_specs=[
            pl.BlockSpec((1, 12, L), lambda b, i: (b, 0, 0)),
            pl.BlockSpec((1, RB, 12), lambda b, i: (b, i, 0)),
            pl.BlockSpec((1, RB, 1), lambda b, i: (b, i, 0)),
            pl.BlockSpec((8, DIM_NODES), lambda b, i: (0, 0)),
            pl.BlockSpec((1, DIM_NODES), lambda b, i: (0, 0)),
        ],
        out_specs=[
            pl.BlockSpec((RB, L), lambda b, i: (b * nblk + i, 0)),
            pl.BlockSpec((1, RB, DIM_NODES), lambda b, i: (b, i, 0)),
            pl.BlockSpec((1, RB, 1), lambda b, i: (b, i, 0)),
        ],
        out_shape=[
            jax.ShapeDtypeStruct((BL, L), F32),
            jax.ShapeDtypeStruct((B, L, DIM_NODES), F32),
            jax.ShapeDtypeStruct((B, L, 1), F32),
        ],
    )(Xt, Xf, C3, Wn_pad, bn2)


def _sc_topk_gather(D, table):
    mesh = plsc.VectorSubcoreMesh(core_axis_name="c", subcore_axis_name="s")

    @functools.partial(
        pl.kernel, mesh=mesh,
        compiler_params=pltpu.CompilerParams(
            use_tc_tiling_on_sc=True, needs_layout_passes=False),
        out_type=[
            jax.ShapeDtypeStruct((BL, 32), jnp.int32),
            jax.ShapeDtypeStruct((BL * 32, 128), F32),
        ],
        scratch_types=[
            pltpu.VMEM((BR, L), F32),
            pltpu.VMEM((BR, L), F32),
            pltpu.VMEM((1056,), F32),
            pltpu.VMEM((1056,), jnp.int32),
            pltpu.VMEM((RPT, 32), jnp.int32),
            pltpu.VMEM((BR * 32,), jnp.int32),
            pltpu.VMEM((BR * 32,), jnp.int32),
            pltpu.VMEM((BR * 32, 128), F32),
            pltpu.VMEM((BR * 32, 128), F32),
            pltpu.SemaphoreType.DMA,
            pltpu.SemaphoreType.DMA,
            pltpu.SemaphoreType.DMA,
            pltpu.SemaphoreType.DMA,
            pltpu.SemaphoreType.DMA,
            pltpu.SemaphoreType.DMA,
        ],
    )
    def k(d_hbm, t_hbm, eidx_hbm, xj_hbm, db0, db1, cval, cidx, ebuf,
          gb0, gb1, xb0, xb1, sd0, sd1, sg0, sg1, so0, so1):
        wid = lax.axis_index("s") * 2 + lax.axis_index("c")
        r0 = wid * RPT
        base = (r0 // L) * L
        lanes = lax.broadcasted_iota(jnp.int32, (16,), 0)
        infv = jnp.full((16,), jnp.inf, F32)
        bigi = jnp.full((16,), jnp.int32(2 ** 30), jnp.int32)
        dbufs = (db0, db1)
        dsems = (sd0, sd1)
        gbufs = (gb0, gb1)
        xbufs = (xb0, xb1)
        gsems = (sg0, sg1)
        osems = (so0, so1)

        def topk_row(r, dbuf):
            def p1(i, carry):
                m1, m2 = carry
                v = dbuf[r, pl.ds(i * 16, 16)]
                m2n = jnp.minimum(m2, jnp.maximum(m1, v))
                m1n = jnp.minimum(m1, v)
                return m1n, m2n
            m1, m2 = lax.fori_loop(0, L // 16, p1, (infv, infv))
            t = jnp.maximum(jnp.max(m1), jnp.max(m2))

            def p2(i, cnt):
                v = dbuf[r, pl.ds(i * 16, 16)]
                msk = v <= t
                idxv = i * 16 + lanes
                plsc.store_compressed(cval.at[pl.ds(cnt, 16)], v, mask=msk)
                plsc.store_compressed(cidx.at[pl.ds(cnt, 16)], idxv, mask=msk)
                return cnt + plsc.all_reduce_population_count(msk)[0]
            cnt = lax.fori_loop(0, L // 16, p2, jnp.int32(0))
            cval[pl.ds(cnt, 16)] = infv
            nv = (cnt + 15) // 16

            own = r0 + r - base

            gdn = lax.GatherDimensionNumbers(
                offset_dims=(), collapsed_slice_dims=(0,),
                start_index_map=(0,))

            def vmin_tree(v):
                for sh in (1, 2, 4, 8):
                    perm = (lanes + sh) % 16
                    sv = lax.gather(
                        v, perm[:, None], gdn, slice_sizes=(1,),
                        mode=lax.GatherScatterMode.PROMISE_IN_BOUNDS)
                    v = jnp.minimum(v, sv)
                return v

            def ext(kk, carry):
                e0, e1, pmi = carry

                def scan(c, sc):
                    bv, bi = sc
                    v = cval[pl.ds(c * 16, 16)]
                    ix = cidx[pl.ds(c * 16, 16)]
                    v = jnp.where(ix == pmi, jnp.inf, v)
                    cval[pl.ds(c * 16, 16)] = v
                    take = (v < bv) | ((v == bv) & (ix < bi))
                    return (jnp.where(take, v, bv), jnp.where(take, ix, bi))
                bv, bi = lax.fori_loop(0, nv, scan, (infv, bigi))
                mv = vmin_tree(bv)
                miv = vmin_tree(jnp.where(bv == mv, bi, bigi))
                e0 = jnp.where(lanes == kk, miv, e0)
                e1 = jnp.where(lanes == kk - 16, miv, e1)
                return e0, e1, miv

            owns = jnp.full((16,), own, jnp.int32)
            e0, e1, _ = lax.fori_loop(0, K, ext, (owns, owns, bigi))
            return e0, e1

        dcp = pltpu.async_copy(d_hbm.at[pl.ds(r0, BR)], db0, sd0)
        outcp = [None, None]
        for bt in range(NBATCH):
            par = bt % 2
            if bt + 1 < NBATCH:
                ncp = pltpu.async_copy(
                    d_hbm.at[pl.ds(r0 + (bt + 1) * BR, BR)],
                    dbufs[(bt + 1) % 2], dsems[(bt + 1) % 2])
            dcp.wait()
            if outcp[par] is not None:
                outcp[par].wait()

            dbuf = dbufs[par]
            gbuf = gbufs[par]

            def row_fn(r, _):
                e0, e1 = topk_row(r, dbuf)
                row = bt * BR + r
                ebuf[row, pl.ds(0, 16)] = e0
                ebuf[row, pl.ds(16, 16)] = e1
                gbuf[pl.ds(r * 32, 16)] = e0 + base
                gbuf[pl.ds(r * 32 + 16, 16)] = e1 + base
                return 0
            lax.fori_loop(0, BR, row_fn, 0)

            g0 = pltpu.async_copy(t_hbm.at[gbuf.at[pl.ds(0, 128)]],
                                  xbufs[par].at[pl.ds(0, 128)], gsems[par])
            g1 = pltpu.async_copy(t_hbm.at[gbuf.at[pl.ds(128, 128)]],
                                  xbufs[par].at[pl.ds(128, 128)], gsems[par])
            g0.wait()
            g1.wait()
            outcp[par] = pltpu.async_copy(
                xbufs[par],
                xj_hbm.at[pl.ds((r0 + bt * BR) * 32, BR * 32)], osems[par])
            if bt + 1 < NBATCH:
                dcp = ncp
        for oc in outcp:
            if oc is not None:
                oc.wait()
        pltpu.sync_copy(ebuf, eidx_hbm.at[pl.ds(r0, RPT)])

    return k(D, table)


def _edge_body(ti_ref, xj_ref, w16_ref, w17_ref, be_ref, eh_ref, mij_ref):
    xi = ti_ref[...]
    xj = xj_ref[...]
    xib = jnp.broadcast_to(xi[:, None, :], (RB2, 32, 128))

    diff = xib[:, :, 0:48] - xj[:, :, 48:96]
    dx = diff[:, :, 0:16]
    dy = diff[:, :, 16:32]
    dz = diff[:, :, 32:48]
    d2 = (dx * dx + dy * dy) + dz * dz
    dcr = jnp.log1p(jnp.sqrt(d2 + 1e-8))
    offv = jnp.tanh((xj[:, :, 97:98] - xib[:, :, 97:98]) * F32(1.0 / 32.0))

    eh = jnp.dot(dcr.reshape(RB2 * 32, 16), w16_ref[...],
                 preferred_element_type=F32).reshape(RB2, 32, DIM_EDGES)
    eh = eh + offv * w17_ref[...].reshape(1, 1, DIM_EDGES)
    eh = eh + be_ref[...].reshape(1, 1, DIM_EDGES)
    mij = xib[:, :, 96:97] * xj[:, :, 96:97]
    eh_ref[...] = (eh * mij)[:, :K, :]
    mij_ref[...] = mij[:, :K, :]


def _encode_edges(table, Xj, W16, W17, be2):
    grid = (BL // RB2,)
    return pl.pallas_call(
        _edge_body,
        grid=grid,
        in_specs=[
            pl.BlockSpec((RB2, 128), lambda i: (i, 0)),
            pl.BlockSpec((RB2, 32, 128), lambda i: (i, 0, 0)),
            pl.BlockSpec((16, DIM_EDGES), lambda i: (0, 0)),
            pl.BlockSpec((1, DIM_EDGES), lambda i: (0, 0)),
            pl.BlockSpec((1, DIM_EDGES), lambda i: (0, 0)),
        ],
        out_specs=[
            pl.BlockSpec((RB2, K, DIM_EDGES), lambda i: (i, 0, 0)),
            pl.BlockSpec((RB2, K, 1), lambda i: (i, 0, 0)),
        ],
        out_shape=[
            jax.ShapeDtypeStruct((BL, K, DIM_EDGES), F32),
            jax.ShapeDtypeStruct((BL, K, 1), F32),
        ],
    )(table, Xj, W16, W17, be2)


def kernel(X, C, W_node, b_node, W_edge, b_edge):
    Xf = X.reshape(B, L, A * 3)
    Xt = jnp.transpose(Xf, (0, 2, 1))
    C3 = C.reshape(B, L, 1)
    Wn_pad = jnp.concatenate([W_node, jnp.zeros((2, DIM_NODES), F32)], axis=0)
    bn2 = b_node.reshape(1, DIM_NODES)

    D, node_h, mask_i3 = _encode_nodes(Xf, Xt, C3, Wn_pad, bn2)
    mask_i = mask_i3.reshape(B, L)

    Xc = jnp.transpose(X, (0, 1, 3, 2))
    ti48 = jnp.broadcast_to(Xc[:, :, :, :, None],
                            (B, L, 3, 4, 4)).reshape(B, L, 48)
    tj48 = jnp.broadcast_to(Xc[:, :, :, None, :],
                            (B, L, 3, 4, 4)).reshape(B, L, 48)
    lvals = jnp.broadcast_to(
        jnp.arange(L, dtype=F32)[None, :, None], (B, L, 1))
    pad = jnp.zeros((B, L, 30), F32)
    table = jnp.concatenate([ti48, tj48, mask_i3, lvals, pad],
                            axis=2).reshape(BL, 128)

    eidx32, xj = _sc_topk_gather(D, table)
    edge_idx = eidx32[:, :K].reshape(B, L, K)
    Xj = xj.reshape(BL, 32, 128)

    W16 = W_edge[0:16]
    W17 = W_edge[16:17]
    be2 = b_edge.reshape(1, DIM_EDGES)
    eh_flat, mij_flat = _encode_edges(table, Xj, W16, W17, be2)

    edge_h = eh_flat.reshape(B, L, K, DIM_EDGES)
    mask_ij = mij_flat.reshape(B, L, K)
    return node_h, edge_h, edge_idx, mask_i, mask_ij


# speedup vs baseline: 1.3540x; 1.1535x over previous
import functools

import jax
import jax.numpy as jnp
from jax import lax
from jax.experimental import pallas as pl
from jax.experimental.pallas import tpu as pltpu
from jax.experimental.pallas import tpu_sc as plsc

B, L, A, K = 4, 1024, 4, 30
BL = B * L
DIM_NODES, DIM_EDGES = 256, 128
RB = 256
RB2 = 64
NW = 32
RPT = BL // NW
BR = 8
NBATCH = RPT // BR
F32 = jnp.float32


def _enc_body(xt_ref, xr_ref, c_ref, wn_ref, bn_ref,
              d_ref, nodeh_ref, maski_ref):
    i = pl.program_id(1)
    xt = xt_ref[0]
    xr = xr_ref[0]

    cax = (((xt[0:1] + xt[3:4]) + xt[6:7]) + xt[9:10]) * 0.25
    cay = (((xt[1:2] + xt[4:5]) + xt[7:8]) + xt[10:11]) * 0.25
    caz = (((xt[2:3] + xt[5:6]) + xt[8:9]) + xt[11:12]) * 0.25
    crx = (((xr[:, 0:1] + xr[:, 3:4]) + xr[:, 6:7]) + xr[:, 9:10]) * 0.25
    cry = (((xr[:, 1:2] + xr[:, 4:5]) + xr[:, 7:8]) + xr[:, 10:11]) * 0.25
    crz = (((xr[:, 2:3] + xr[:, 5:6]) + xr[:, 8:9]) + xr[:, 11:12]) * 0.25

    dx = crx - cax
    dy = cry - cay
    dz = crz - caz
    dist = jnp.sqrt((dx * dx + dy * dy) + dz * dz + 1e-8)
    colj = lax.broadcasted_iota(jnp.int32, (RB, L), 1)
    rowg = i * RB + lax.broadcasted_iota(jnp.int32, (RB, L), 0)
    d_ref[...] = dist + jnp.where(colj == rowg, F32(1e6), F32(0.0))

    cols = []
    for (a, b) in ((0, 1), (0, 2), (0, 3), (1, 2), (1, 3), (2, 3)):
        pdx = xr[:, 3 * a:3 * a + 1] - xr[:, 3 * b:3 * b + 1]
        pdy = xr[:, 3 * a + 1:3 * a + 2] - xr[:, 3 * b + 1:3 * b + 2]
        pdz = xr[:, 3 * a + 2:3 * a + 3] - xr[:, 3 * b + 2:3 * b + 3]
        pd2 = (pdx * pdx + pdy * pdy) + pdz * pdz
        cols.append(jnp.log1p(jnp.sqrt(pd2 + 1e-8)))
    nf = jnp.concatenate(cols + [jnp.zeros((RB, 2), F32)], axis=1)

    mask = (c_ref[0] > 0).astype(F32)
    nh = jnp.dot(nf, wn_ref[...], preferred_element_type=F32) + bn_ref[...]
    nodeh_ref[0] = nh * mask
    maski_ref[0] = mask


def _encode_nodes(Xf, Xt, C3, Wn_pad, bn2):
    grid = (B, L // RB)
    nblk = L // RB
    return pl.pallas_call(
        _enc_body,
        grid=grid,
        in_specs=[
            pl.BlockSpec((1, 12, L), lambda b, i: (b, 0, 0)),
            pl.BlockSpec((1, RB, 12), lambda b, i: (b, i, 0)),
            pl.BlockSpec((1, RB, 1), lambda b, i: (b, i, 0)),
            pl.BlockSpec((8, DIM_NODES), lambda b, i: (0, 0)),
            pl.BlockSpec((1, DIM_NODES), lambda b, i: (0, 0)),
        ],
        out_specs=[
            pl.BlockSpec((RB, L), lambda b, i: (b * nblk + i, 0)),
            pl.BlockSpec((1, RB, DIM_NODES), lambda b, i: (b, i, 0)),
            pl.BlockSpec((1, RB, 1), lambda b, i: (b, i, 0)),
        ],
        out_shape=[
            jax.ShapeDtypeStruct((BL, L), F32),
            jax.ShapeDtypeStruct((B, L, DIM_NODES), F32),
            jax.ShapeDtypeStruct((B, L, 1), F32),
        ],
    )(Xt, Xf, C3, Wn_pad, bn2)


def _sc_topk_gather(D, table):
    mesh = plsc.VectorSubcoreMesh(core_axis_name="c", subcore_axis_name="s")

    @functools.partial(
        pl.kernel, mesh=mesh,
        compiler_params=pltpu.CompilerParams(
            use_tc_tiling_on_sc=True, needs_layout_passes=False),
        out_type=[
            jax.ShapeDtypeStruct((BL, 32), jnp.int32),
            jax.ShapeDtypeStruct((BL * 32, 128), F32),
        ],
        scratch_types=[
            pltpu.VMEM((BR, L), F32),
            pltpu.VMEM((BR, L), F32),
            pltpu.VMEM((1056,), F32),
            pltpu.VMEM((1056,), jnp.int32),
            pltpu.VMEM((RPT, 32), jnp.int32),
            pltpu.VMEM((BR * 32,), jnp.int32),
            pltpu.VMEM((BR * 32,), jnp.int32),
            pltpu.VMEM((BR * 32, 128), F32),
            pltpu.VMEM((BR * 32, 128), F32),
            pltpu.SemaphoreType.DMA,
            pltpu.SemaphoreType.DMA,
            pltpu.SemaphoreType.DMA,
            pltpu.SemaphoreType.DMA,
            pltpu.SemaphoreType.DMA,
            pltpu.SemaphoreType.DMA,
        ],
    )
    def k(d_hbm, t_hbm, eidx_hbm, xj_hbm, db0, db1, cval, cidx, ebuf,
          gb0, gb1, xb0, xb1, sd0, sd1, sg0, sg1, so0, so1):
        wid = lax.axis_index("s") * 2 + lax.axis_index("c")
        r0 = wid * RPT
        base = (r0 // L) * L
        lanes = lax.broadcasted_iota(jnp.int32, (16,), 0)
        infv = jnp.full((16,), jnp.inf, F32)
        bigi = jnp.full((16,), jnp.int32(2 ** 30), jnp.int32)
        dbufs = (db0, db1)
        dsems = (sd0, sd1)
        gbufs = (gb0, gb1)
        xbufs = (xb0, xb1)
        gsems = (sg0, sg1)
        osems = (so0, so1)

        def topk_row(r, dbuf):
            def p1(i, carry):
                m1, m2 = carry
                for u in range(4):
                    v = dbuf[r, pl.ds((i * 4 + u) * 16, 16)]
                    m2 = jnp.minimum(m2, jnp.maximum(m1, v))
                    m1 = jnp.minimum(m1, v)
                return m1, m2
            m1, m2 = lax.fori_loop(0, L // 64, p1, (infv, infv))
            t = jnp.maximum(jnp.max(m1), jnp.max(m2))

            def p2(i, cnt):
                v = dbuf[r, pl.ds(i * 16, 16)]
                msk = v <= t
                idxv = i * 16 + lanes
                plsc.store_compressed(cval.at[pl.ds(cnt, 16)], v, mask=msk)
                plsc.store_compressed(cidx.at[pl.ds(cnt, 16)], idxv, mask=msk)
                return cnt + plsc.all_reduce_population_count(msk)[0]
            cnt = lax.fori_loop(0, L // 16, p2, jnp.int32(0))
            cval[pl.ds(cnt, 16)] = infv
            nv = (cnt + 15) // 16

            own = r0 + r - base

            gdn = lax.GatherDimensionNumbers(
                offset_dims=(), collapsed_slice_dims=(0,),
                start_index_map=(0,))

            def vmin_tree(v):
                for sh in (1, 2, 4, 8):
                    perm = (lanes + sh) % 16
                    sv = lax.gather(
                        v, perm[:, None], gdn, slice_sizes=(1,),
                        mode=lax.GatherScatterMode.PROMISE_IN_BOUNDS)
                    v = jnp.minimum(v, sv)
                return v

            def ext(kk, carry):
                e0, e1, pA, pB = carry

                def scan(c, sc):
                    bv1, bi1, bv2, bi2 = sc
                    v = cval[pl.ds(c * 16, 16)]
                    ix = cidx[pl.ds(c * 16, 16)]
                    v = jnp.where((ix == pA) | (ix == pB), jnp.inf, v)
                    cval[pl.ds(c * 16, 16)] = v
                    lt1 = (v < bv1) | ((v == bv1) & (ix < bi1))
                    lt2 = (v < bv2) | ((v == bv2) & (ix < bi2))
                    bv2n = jnp.where(lt1, bv1, jnp.where(lt2, v, bv2))
                    bi2n = jnp.where(lt1, bi1, jnp.where(lt2, ix, bi2))
                    return (jnp.where(lt1, v, bv1), jnp.where(lt1, ix, bi1),
                            bv2n, bi2n)
                bv1, bi1, bv2, bi2 = lax.fori_loop(
                    0, nv, scan, (infv, bigi, infv, bigi))
                mA = vmin_tree(bv1)
                miA = vmin_tree(jnp.where(bv1 == mA, bi1, bigi))
                isA = (bv1 == mA) & (bi1 == miA)
                v2 = jnp.where(isA, bv2, bv1)
                i2 = jnp.where(isA, bi2, bi1)
                mB = vmin_tree(v2)
                miB = vmin_tree(jnp.where(v2 == mB, i2, bigi))
                kk2 = 2 * kk
                e0 = jnp.where(lanes == kk2, miA, e0)
                e1 = jnp.where(lanes == kk2 - 16, miA, e1)
                e0 = jnp.where(lanes == kk2 + 1, miB, e0)
                e1 = jnp.where(lanes == kk2 - 15, miB, e1)
                return e0, e1, miA, miB

            owns = jnp.full((16,), own, jnp.int32)
            e0, e1, _, _ = lax.fori_loop(0, K // 2, ext,
                                         (owns, owns, bigi, bigi))
            return e0, e1

        dcp = pltpu.async_copy(d_hbm.at[pl.ds(r0, BR)], db0, sd0)
        outcp = [None, None]
        for bt in range(NBATCH):
            par = bt % 2
            if bt + 1 < NBATCH:
                ncp = pltpu.async_copy(
                    d_hbm.at[pl.ds(r0 + (bt + 1) * BR, BR)],
                    dbufs[(bt + 1) % 2], dsems[(bt + 1) % 2])
            dcp.wait()
            if outcp[par] is not None:
                outcp[par].wait()

            dbuf = dbufs[par]
            gbuf = gbufs[par]

            def row_fn(r, _):
                e0, e1 = topk_row(r, dbuf)
                row = bt * BR + r
                ebuf[row, pl.ds(0, 16)] = e0
                ebuf[row, pl.ds(16, 16)] = e1
                gbuf[pl.ds(r * 32, 16)] = e0 + base
                gbuf[pl.ds(r * 32 + 16, 16)] = e1 + base
                return 0
            lax.fori_loop(0, BR, row_fn, 0)

            g0 = pltpu.async_copy(t_hbm.at[gbuf.at[pl.ds(0, 128)]],
                                  xbufs[par].at[pl.ds(0, 128)], gsems[par])
            g1 = pltpu.async_copy(t_hbm.at[gbuf.at[pl.ds(128, 128)]],
                                  xbufs[par].at[pl.ds(128, 128)], gsems[par])
            g0.wait()
            g1.wait()
            outcp[par] = pltpu.async_copy(
                xbufs[par],
                xj_hbm.at[pl.ds((r0 + bt * BR) * 32, BR * 32)], osems[par])
            if bt + 1 < NBATCH:
                dcp = ncp
        for oc in outcp:
            if oc is not None:
                oc.wait()
        pltpu.sync_copy(ebuf, eidx_hbm.at[pl.ds(r0, RPT)])

    return k(D, table)


def _edge_body(ti_ref, xj_ref, w16_ref, w17_ref, be_ref, eh_ref, mij_ref):
    xi = ti_ref[...]
    xj = xj_ref[...].reshape(RB2, 32, 128)
    xib = jnp.broadcast_to(xi[:, None, :], (RB2, 32, 128))

    diff = xib[:, :, 0:48] - xj[:, :, 48:96]
    dx = diff[:, :, 0:16]
    dy = diff[:, :, 16:32]
    dz = diff[:, :, 32:48]
    d2 = (dx * dx + dy * dy) + dz * dz
    dcr = jnp.log1p(jnp.sqrt(d2 + 1e-8))
    offv = jnp.tanh((xj[:, :, 97:98] - xib[:, :, 97:98]) * F32(1.0 / 32.0))

    eh = jnp.dot(dcr.reshape(RB2 * 32, 16), w16_ref[...],
                 preferred_element_type=F32).reshape(RB2, 32, DIM_EDGES)
    eh = eh + offv * w17_ref[...].reshape(1, 1, DIM_EDGES)
    eh = eh + be_ref[...].reshape(1, 1, DIM_EDGES)
    mij = xib[:, :, 96:97] * xj[:, :, 96:97]
    eh_ref[...] = (eh * mij)[:, :K, :]
    mij_ref[...] = mij[:, :K, :]


def _encode_edges(table, Xj, W16, W17, be2):
    grid = (BL // RB2,)
    return pl.pallas_call(
        _edge_body,
        grid=grid,
        in_specs=[
            pl.BlockSpec((RB2, 128), lambda i: (i, 0)),
            pl.BlockSpec((RB2 * 32, 128), lambda i: (i, 0)),
            pl.BlockSpec((16, DIM_EDGES), lambda i: (0, 0)),
            pl.BlockSpec((1, DIM_EDGES), lambda i: (0, 0)),
            pl.BlockSpec((1, DIM_EDGES), lambda i: (0, 0)),
        ],
        out_specs=[
            pl.BlockSpec((RB2, K, DIM_EDGES), lambda i: (i, 0, 0)),
            pl.BlockSpec((RB2, K, 1), lambda i: (i, 0, 0)),
        ],
        out_shape=[
            jax.ShapeDtypeStruct((BL, K, DIM_EDGES), F32),
            jax.ShapeDtypeStruct((BL, K, 1), F32),
        ],
    )(table, Xj, W16, W17, be2)


def kernel(X, C, W_node, b_node, W_edge, b_edge):
    Xf = X.reshape(B, L, A * 3)
    Xt = jnp.transpose(Xf, (0, 2, 1))
    C3 = C.reshape(B, L, 1)
    Wn_pad = jnp.concatenate([W_node, jnp.zeros((2, DIM_NODES), F32)], axis=0)
    bn2 = b_node.reshape(1, DIM_NODES)

    D, node_h, mask_i3 = _encode_nodes(Xf, Xt, C3, Wn_pad, bn2)
    mask_i = mask_i3.reshape(B, L)

    Xc = jnp.transpose(X, (0, 1, 3, 2))
    ti48 = jnp.broadcast_to(Xc[:, :, :, :, None],
                            (B, L, 3, 4, 4)).reshape(B, L, 48)
    tj48 = jnp.broadcast_to(Xc[:, :, :, None, :],
                            (B, L, 3, 4, 4)).reshape(B, L, 48)
    lvals = jnp.broadcast_to(
        jnp.arange(L, dtype=F32)[None, :, None], (B, L, 1))
    pad = jnp.zeros((B, L, 30), F32)
    table = jnp.concatenate([ti48, tj48, mask_i3, lvals, pad],
                            axis=2).reshape(BL, 128)

    eidx32, Xj = _sc_topk_gather(D, table)
    edge_idx = eidx32[:, :K].reshape(B, L, K)

    W16 = W_edge[0:16]
    W17 = W_edge[16:17]
    be2 = b_edge.reshape(1, DIM_EDGES)
    eh_flat, mij_flat = _encode_edges(table, Xj, W16, W17, be2)

    edge_h = eh_flat.reshape(B, L, K, DIM_EDGES)
    mask_ij = mij_flat.reshape(B, L, K)
    return node_h, edge_h, edge_idx, mask_i, mask_ij
